# baseline (speedup 1.0000x reference)
import functools

import jax
import jax.numpy as jnp
from jax.experimental import pallas as pl
from jax.experimental.pallas import tpu as pltpu

CS = 72
ROWW = CS * 3
BB = 32


def _paste_body(pos_ref, col_ref, out_ref):
    r_iota = jax.lax.broadcasted_iota(jnp.int32, (CS, ROWW), 0)
    w_iota = jax.lax.broadcasted_iota(jnp.int32, (CS, ROWW), 1)
    ch = w_iota % 3

    def one(s, carry):
        r_lo = pos_ref[s, 0]
        r_hi = pos_ref[s, 1]
        c_lo3 = pos_ref[s, 2]
        c_hi3 = pos_ref[s, 3]
        mask = (
            (r_iota >= r_lo)
            & (r_iota < r_hi)
            & (w_iota >= c_lo3)
            & (w_iota < c_hi3)
        )
        colsel = jnp.where(
            ch == 0,
            col_ref[s, 0],
            jnp.where(ch == 1, col_ref[s, 1], col_ref[s, 2]),
        )
        out_ref[s] = jnp.where(mask, colsel, jnp.float32(255.0))
        return carry

    jax.lax.fori_loop(0, BB, one, 0)


@jax.jit
def kernel(positions, colors):
    pos = positions.astype(jnp.int32)
    r_lo = jnp.minimum(pos[:, 0, 0], CS)
    r_hi = jnp.minimum(pos[:, 1, 0], CS)
    c_lo = jnp.minimum(pos[:, 0, 1], CS)
    c_hi = jnp.minimum(pos[:, 1, 1], CS)
    pos4 = jnp.stack([r_lo, r_hi, c_lo * 3, c_hi * 3], axis=1)
    b = pos4.shape[0]

    out = pl.pallas_call(
        _paste_body,
        grid=(b // BB,),
        in_specs=[
            pl.BlockSpec((BB, 4), lambda i: (i, 0), memory_space=pltpu.SMEM),
            pl.BlockSpec((BB, 3), lambda i: (i, 0), memory_space=pltpu.SMEM),
        ],
        out_specs=pl.BlockSpec((BB, CS, ROWW), lambda i: (i, 0, 0)),
        out_shape=jax.ShapeDtypeStruct((b, CS, ROWW), jnp.float32),
    )(pos4, colors)
    return out.reshape(b, CS, CS, 3)
